# Initial kernel scaffold; baseline (speedup 1.0000x reference)
import jax
import jax.numpy as jnp
from jax.experimental import pallas as pl


def kernel(input_tensor, weight):
    raise NotImplementedError("write your pallas kernel here")


# speedup vs baseline: 2.6554x; 2.6554x over previous
import functools

import jax
import jax.numpy as jnp
from jax import lax
from jax.experimental import pallas as pl
from jax.experimental.pallas import tpu as pltpu
from jax.experimental.pallas import tpu_sc as plsc

B0, B1 = 16384, 200
NUM_E, DIM = 10, 3
TOTAL_WORDS = B0 * B1 * DIM // 2
NUM_WORKERS = 32
WPW = TOTAL_WORDS // NUM_WORKERS
NCHUNK = 10
CW = WPW // NCHUNK
IW = CW * 2 // 3
GROUPS = CW // 48


def _body(idx_hbm, wi_hbm, out_hbm, idx_v, out_v, pt_v, wi_v):
    wid = lax.axis_index("s") * 2 + lax.axis_index("c")
    word_base = wid * WPW
    idx_base = wid * (WPW * 2 // 3)

    pltpu.sync_copy(wi_hbm, wi_v)
    lanes = lax.broadcasted_iota(jnp.int32, (16,), 0)
    for t in range(7):
        c = jnp.minimum(lanes + 16 * t, 99)
        a = c // 10
        b = c - 10 * a
        wa0 = plsc.load_gather(wi_v, [3 * a])
        wa1 = plsc.load_gather(wi_v, [3 * a + 1])
        wa2 = plsc.load_gather(wi_v, [3 * a + 2])
        wb0 = plsc.load_gather(wi_v, [3 * b])
        wb1 = plsc.load_gather(wi_v, [3 * b + 1])
        wb2 = plsc.load_gather(wi_v, [3 * b + 2])
        plsc.store_scatter(pt_v, [3 * c], wa0 | (wa1 << 16))
        plsc.store_scatter(pt_v, [3 * c + 1], wa2 | (wb0 << 16))
        plsc.store_scatter(pt_v, [3 * c + 2], wb1 | (wb2 << 16))

    A = []
    R = []
    for i in range(3):
        e = lanes + 16 * i
        q = e // 3
        A.append(2 * q)
        R.append(e - 3 * q)

    def group(g, _):
        base = 32 * g
        for i in range(3):
            va = plsc.load_gather(idx_v, [A[i] + base])
            vb = plsc.load_gather(idx_v, [A[i] + (base + 1)])
            w = plsc.load_gather(pt_v, [va * 30 + vb * 3 + R[i]])
            out_v[pl.ds(48 * g + 16 * i, 16)] = w
        return _

    for chunk in range(NCHUNK):
        pltpu.sync_copy(idx_hbm.at[pl.ds(idx_base + IW * chunk, IW)], idx_v)
        lax.fori_loop(0, GROUPS, group, None, unroll=2)
        pltpu.sync_copy(out_v, out_hbm.at[pl.ds(word_base + CW * chunk, CW)])


_mesh = plsc.VectorSubcoreMesh(core_axis_name="c", subcore_axis_name="s")

_sc_call = functools.partial(
    pl.kernel,
    mesh=_mesh,
    out_type=jax.ShapeDtypeStruct((TOTAL_WORDS,), jnp.int32),
    scratch_types=[
        pltpu.VMEM((IW,), jnp.int32),
        pltpu.VMEM((CW,), jnp.int32),
        pltpu.VMEM((304,), jnp.int32),
        pltpu.VMEM((32,), jnp.int32),
    ],
    compiler_params=pltpu.CompilerParams(needs_layout_passes=False),
)(_body)


@jax.jit
def kernel(input_tensor, weight):
    idx_flat = input_tensor.reshape(-1)
    w16 = jax.lax.bitcast_convert_type(weight, jnp.uint16).reshape(-1)
    wi = jnp.zeros((32,), jnp.int32).at[:30].set(w16.astype(jnp.int32))
    out_words = _sc_call(idx_flat, wi)
    out = jax.lax.bitcast_convert_type(out_words, jnp.float16)
    return out.reshape(B0, B1, DIM)


# speedup vs baseline: 55.9623x; 21.0745x over previous
import functools

import jax
import jax.numpy as jnp
from jax import lax
from jax.experimental import pallas as pl
from jax.experimental.pallas import tpu as pltpu
from jax.experimental.pallas import tpu_sc as plsc

B0, B1 = 16384, 200
NUM_E, DIM = 10, 3
TOTAL_WORDS = B0 * B1 * DIM // 2
PLANE = B0 * B1 // 2
NJB = B1 // 8
NIB = B0 // 128
IBW = NIB // 32


def _body(idx_hbm, wi_hbm, out_hbm, in_v, out_v, pt_v, wi_v):
    wid = lax.axis_index("s") * 2 + lax.axis_index("c")

    pltpu.sync_copy(wi_hbm, wi_v)
    lanes = lax.broadcasted_iota(jnp.int32, (16,), 0)
    for t in range(7):
        c = jnp.minimum(lanes + 16 * t, 99)
        a = c // 10
        b = c - 10 * a
        for d in range(3):
            wa = plsc.load_gather(wi_v, [3 * a + d])
            wb = plsc.load_gather(wi_v, [3 * b + d])
            plsc.store_scatter(pt_v, [4 * c + d], wa | (wb << 16))

    def jb_step(jb, _):
        in_off = jb * (NIB * 1024) + wid * (IBW * 1024)
        pltpu.sync_copy(idx_hbm.at[pl.ds(in_off, IBW * 1024)], in_v)
        for t in range(IBW):
            for jq in range(4):
                for k in range(8):
                    va = in_v[pl.ds(t * 1024 + (2 * jq) * 128 + 16 * k, 16)]
                    vb = in_v[pl.ds(t * 1024 + (2 * jq + 1) * 128 + 16 * k, 16)]
                    c = (va * 10 + vb) * 4
                    dst = t * 512 + jq * 128 + 16 * k
                    for d in range(3):
                        w = plsc.load_gather(pt_v, [c + d])
                        out_v[pl.ds(d * (IBW * 512) + dst, 16)] = w
        out_off = jb * (NIB * 512) + wid * (IBW * 512)
        for d in range(3):
            pltpu.sync_copy(
                out_v.at[pl.ds(d * (IBW * 512), IBW * 512)],
                out_hbm.at[pl.ds(d * PLANE + out_off, IBW * 512)],
            )
        return _

    lax.fori_loop(0, NJB, jb_step, None)


_mesh = plsc.VectorSubcoreMesh(core_axis_name="c", subcore_axis_name="s")

_sc_call = functools.partial(
    pl.kernel,
    mesh=_mesh,
    out_type=jax.ShapeDtypeStruct((TOTAL_WORDS,), jnp.int32),
    scratch_types=[
        pltpu.VMEM((IBW * 1024,), jnp.int32),
        pltpu.VMEM((3 * IBW * 512,), jnp.int32),
        pltpu.VMEM((400,), jnp.int32),
        pltpu.VMEM((32,), jnp.int32),
    ],
    compiler_params=pltpu.CompilerParams(needs_layout_passes=False),
)(_body)


@jax.jit
def kernel(input_tensor, weight):
    idx_flat = (
        input_tensor.reshape(128, 128, 25, 8)
        .transpose(2, 0, 3, 1)
        .reshape(-1)
    )
    w16 = jax.lax.bitcast_convert_type(weight, jnp.uint16).reshape(-1)
    wi = jnp.zeros((32,), jnp.int32).at[:30].set(w16.astype(jnp.int32))
    out_words = _sc_call(idx_flat, wi)
    pairs = jax.lax.bitcast_convert_type(out_words, jnp.float16)
    out = (
        pairs.reshape(3, 25, 128, 4, 128, 2)
        .transpose(2, 4, 1, 3, 5, 0)
        .reshape(B0, B1, DIM)
    )
    return out


# speedup vs baseline: 180.2378x; 3.2207x over previous
import functools

import jax
import jax.numpy as jnp
from jax import lax
from jax.experimental import pallas as pl
from jax.experimental.pallas import tpu as pltpu
from jax.experimental.pallas import tpu_sc as plsc

B0, B1 = 16384, 200
NUM_E, DIM = 10, 3
TOTAL_WORDS = B0 * B1 * DIM // 2
PLANE = B0 * B1 // 2
NJB = B1 // 8
NIB = B0 // 128
IBW = NIB // 32


def _body(idx_hbm, wi_hbm, out_hbm, in_v, out_v, pt_v, wi_v):
    out_w = out_hbm.bitcast(jnp.int32)
    wid = lax.axis_index("s") * 2 + lax.axis_index("c")

    pltpu.sync_copy(wi_hbm, wi_v)
    lanes = lax.broadcasted_iota(jnp.int32, (16,), 0)
    for t in range(7):
        c = jnp.minimum(lanes + 16 * t, 99)
        a = c // 10
        b = c - 10 * a
        for d in range(3):
            wa = plsc.load_gather(wi_v, [3 * a + d])
            wb = plsc.load_gather(wi_v, [3 * b + d])
            plsc.store_scatter(pt_v, [4 * c + d], wa | (wb << 16))

    def jb_step(jb, _):
        in_off = jb * (NIB * 1024) + wid * (IBW * 1024)
        pltpu.sync_copy(idx_hbm.at[pl.ds(in_off, IBW * 1024)], in_v)
        for t in range(IBW):
            for jq in range(4):
                vas = [
                    in_v[pl.ds(t * 1024 + (2 * jq) * 128 + 16 * k, 16)]
                    for k in range(8)
                ]
                vbs = [
                    in_v[pl.ds(t * 1024 + (2 * jq + 1) * 128 + 16 * k, 16)]
                    for k in range(8)
                ]
                cs = [(vas[k] * 10 + vbs[k]) * 4 for k in range(8)]
                ws = [
                    [plsc.load_gather(pt_v, [cs[k] + d]) for d in range(3)]
                    for k in range(8)
                ]
                for d in range(3):
                    for k in range(8):
                        out_v[d * 16 + t * 4 + jq, pl.ds(16 * k, 16)] = ws[k][d]
        out_row = jb * (NIB * 4) + wid * (IBW * 4)
        for d in range(3):
            pltpu.sync_copy(
                out_v.at[pl.ds(d * 16, IBW * 4), :],
                out_w.at[pl.ds(d * (PLANE // 128) + out_row, IBW * 4), :],
            )
        return _

    lax.fori_loop(0, NJB, jb_step, None)


_mesh = plsc.VectorSubcoreMesh(core_axis_name="c", subcore_axis_name="s")

_sc_call = functools.partial(
    pl.kernel,
    mesh=_mesh,
    out_type=jax.ShapeDtypeStruct((2 * TOTAL_WORDS // 128, 128), jnp.float16),
    scratch_types=[
        pltpu.VMEM((IBW * 1024,), jnp.int32),
        pltpu.VMEM((3 * IBW * 4, 128), jnp.int32),
        pltpu.VMEM((400,), jnp.int32),
        pltpu.VMEM((32,), jnp.int32),
    ],
    compiler_params=pltpu.CompilerParams(needs_layout_passes=False),
)(_body)


@jax.jit
def kernel(input_tensor, weight):
    idx_flat = (
        input_tensor.reshape(128, 128, 25, 8)
        .transpose(2, 0, 3, 1)
        .reshape(-1)
    )
    w16 = jax.lax.bitcast_convert_type(weight, jnp.uint16).reshape(-1)
    wi = jnp.zeros((32,), jnp.int32).at[:30].set(w16.astype(jnp.int32))
    out_halves = _sc_call(idx_flat, wi)
    out = (
        out_halves.reshape(3, 25, 128, 4, 2, 128)
        .transpose(2, 5, 1, 3, 4, 0)
        .reshape(B0, B1, DIM)
    )
    return out


# speedup vs baseline: 216.5356x; 1.2014x over previous
import functools

import jax
import jax.numpy as jnp
from jax import lax
from jax.experimental import pallas as pl
from jax.experimental.pallas import tpu as pltpu
from jax.experimental.pallas import tpu_sc as plsc

B0, B1 = 16384, 200
NUM_E, DIM = 10, 3
TOTAL_WORDS = B0 * B1 * DIM // 2
PLANE = B0 * B1 // 2
NJB = B1 // 8
NIB = B0 // 128
IBW = NIB // 32


def _body(idx_hbm, wi_hbm, out_hbm, in_v, out_v, pt_v, wi_v, sem_in, sem_out):
    out_w = out_hbm.bitcast(jnp.int32)
    wid = lax.axis_index("s") * 2 + lax.axis_index("c")

    pltpu.sync_copy(wi_hbm, wi_v)
    lanes = lax.broadcasted_iota(jnp.int32, (16,), 0)
    for t in range(7):
        c = jnp.minimum(lanes + 16 * t, 99)
        a = c // 10
        b = c - 10 * a
        for d in range(3):
            wa = plsc.load_gather(wi_v, [3 * a + d])
            wb = plsc.load_gather(wi_v, [3 * b + d])
            plsc.store_scatter(pt_v, [4 * c + d], wa | (wb << 16))

    CHUNK = IBW * 1024

    def in_desc(jb, buf):
        in_off = jb * (NIB * 1024) + wid * CHUNK
        return pltpu.make_async_copy(
            idx_hbm.at[pl.ds(in_off, CHUNK)],
            in_v.at[pl.ds(buf * CHUNK, CHUNK)],
            sem_in.at[buf],
        )

    def out_desc(jb, buf, d):
        out_row = jb * (NIB * 4) + wid * (IBW * 4)
        return pltpu.make_async_copy(
            out_v.at[pl.ds(buf * 48 + d * 16, IBW * 4), :],
            out_w.at[pl.ds(d * (PLANE // 128) + out_row, IBW * 4), :],
            sem_out.at[buf],
        )

    def compute(buf):
        ibase = buf * CHUNK
        for t in range(IBW):
            for jq in range(4):
                vas = [
                    in_v[pl.ds(ibase + t * 1024 + (2 * jq) * 128 + 16 * k, 16)]
                    for k in range(8)
                ]
                vbs = [
                    in_v[
                        pl.ds(ibase + t * 1024 + (2 * jq + 1) * 128 + 16 * k, 16)
                    ]
                    for k in range(8)
                ]
                cs = [(vas[k] * 10 + vbs[k]) * 4 for k in range(8)]
                ws = [
                    [plsc.load_gather(pt_v, [cs[k] + d]) for d in range(3)]
                    for k in range(8)
                ]
                for d in range(3):
                    for k in range(8):
                        out_v[buf * 48 + d * 16 + t * 4 + jq, pl.ds(16 * k, 16)] = (
                            ws[k][d]
                        )

    in_desc(0, 0).start()

    def pair_step(p, _):
        jb0 = 2 * p
        jb1 = jb0 + 1
        in_desc(jb0, 0).wait()
        in_desc(jb1, 1).start()

        @pl.when(p >= 1)
        def _drain0():
            for d in range(3):
                out_desc(jb0 - 2, 0, d).wait()

        compute(0)
        for d in range(3):
            out_desc(jb0, 0, d).start()
        in_desc(jb1, 1).wait()
        in_desc(jb0 + 2, 0).start()

        @pl.when(p >= 1)
        def _drain1():
            for d in range(3):
                out_desc(jb1 - 2, 1, d).wait()

        compute(1)
        for d in range(3):
            out_desc(jb1, 1, d).start()
        return _

    lax.fori_loop(0, (NJB - 1) // 2, pair_step, None)

    in_desc(NJB - 1, 0).wait()
    for d in range(3):
        out_desc(NJB - 3, 0, d).wait()
    compute(0)
    for d in range(3):
        out_desc(NJB - 1, 0, d).start()
    for d in range(3):
        out_desc(NJB - 2, 1, d).wait()
        out_desc(NJB - 1, 0, d).wait()


_mesh = plsc.VectorSubcoreMesh(core_axis_name="c", subcore_axis_name="s")

_sc_call = functools.partial(
    pl.kernel,
    mesh=_mesh,
    out_type=jax.ShapeDtypeStruct((2 * TOTAL_WORDS // 128, 128), jnp.float16),
    scratch_types=[
        pltpu.VMEM((2 * IBW * 1024,), jnp.int32),
        pltpu.VMEM((2 * 3 * IBW * 4, 128), jnp.int32),
        pltpu.VMEM((400,), jnp.int32),
        pltpu.VMEM((32,), jnp.int32),
        pltpu.SemaphoreType.DMA((2,)),
        pltpu.SemaphoreType.DMA((2,)),
    ],
    compiler_params=pltpu.CompilerParams(needs_layout_passes=False),
)(_body)


@jax.jit
def kernel(input_tensor, weight):
    idx_flat = (
        input_tensor.reshape(128, 128, 25, 8)
        .transpose(2, 0, 3, 1)
        .reshape(-1)
    )
    w16 = jax.lax.bitcast_convert_type(weight, jnp.uint16).reshape(-1)
    wi = jnp.zeros((32,), jnp.int32).at[:30].set(w16.astype(jnp.int32))
    out_halves = _sc_call(idx_flat, wi)
    out = (
        out_halves.reshape(3, 25, 128, 4, 2, 128)
        .transpose(2, 5, 1, 3, 4, 0)
        .reshape(B0, B1, DIM)
    )
    return out


# speedup vs baseline: 217.0095x; 1.0022x over previous
import functools

import jax
import jax.numpy as jnp
from jax import lax
from jax.experimental import pallas as pl
from jax.experimental.pallas import tpu as pltpu
from jax.experimental.pallas import tpu_sc as plsc

B0, B1 = 16384, 200
NUM_E, DIM = 10, 3
TOTAL_WORDS = B0 * B1 * DIM // 2
PLANE = B0 * B1 // 2
NJB = B1 // 8
NIB = B0 // 128
IBW = NIB // 32


def _body(
    idx_hbm, wi_hbm, out_hbm, in_v, out_v, pt0_v, pt1_v, pt2_v, wi_v, sem_in, sem_out
):
    out_w = out_hbm.bitcast(jnp.int32)
    wid = lax.axis_index("s") * 2 + lax.axis_index("c")
    pts = (pt0_v, pt1_v, pt2_v)

    pltpu.sync_copy(wi_hbm, wi_v)
    lanes = lax.broadcasted_iota(jnp.int32, (16,), 0)
    for t in range(7):
        c = jnp.minimum(lanes + 16 * t, 99)
        a = c // 10
        b = c - 10 * a
        for d in range(3):
            wa = plsc.load_gather(wi_v, [3 * a + d])
            wb = plsc.load_gather(wi_v, [3 * b + d])
            plsc.store_scatter(pts[d], [c], wa | (wb << 16))

    CHUNK = IBW * 1024

    def in_desc(jb, buf):
        in_off = jb * (NIB * 1024) + wid * CHUNK
        return pltpu.make_async_copy(
            idx_hbm.at[pl.ds(in_off, CHUNK)],
            in_v.at[pl.ds(buf * CHUNK, CHUNK)],
            sem_in.at[buf],
        )

    def out_desc(jb, buf, d):
        out_row = jb * (NIB * 4) + wid * (IBW * 4)
        return pltpu.make_async_copy(
            out_v.at[pl.ds(buf * 48 + d * 16, IBW * 4), :],
            out_w.at[pl.ds(d * (PLANE // 128) + out_row, IBW * 4), :],
            sem_out.at[buf],
        )

    def compute(buf):
        ibase = buf * CHUNK
        for t in range(IBW):
            for jq in range(4):
                vas = [
                    in_v[pl.ds(ibase + t * 1024 + (2 * jq) * 128 + 16 * k, 16)]
                    for k in range(8)
                ]
                vbs = [
                    in_v[
                        pl.ds(ibase + t * 1024 + (2 * jq + 1) * 128 + 16 * k, 16)
                    ]
                    for k in range(8)
                ]
                cs = [vas[k] * 10 + vbs[k] for k in range(8)]
                ws = [
                    [plsc.load_gather(pts[d], [cs[k]]) for d in range(3)]
                    for k in range(8)
                ]
                for d in range(3):
                    for k in range(8):
                        out_v[buf * 48 + d * 16 + t * 4 + jq, pl.ds(16 * k, 16)] = (
                            ws[k][d]
                        )

    in_desc(0, 0).start()

    def pair_step(p, _):
        jb0 = 2 * p
        jb1 = jb0 + 1
        in_desc(jb0, 0).wait()
        in_desc(jb1, 1).start()

        @pl.when(p >= 1)
        def _drain0():
            for d in range(3):
                out_desc(jb0 - 2, 0, d).wait()

        compute(0)
        for d in range(3):
            out_desc(jb0, 0, d).start()
        in_desc(jb1, 1).wait()
        in_desc(jb0 + 2, 0).start()

        @pl.when(p >= 1)
        def _drain1():
            for d in range(3):
                out_desc(jb1 - 2, 1, d).wait()

        compute(1)
        for d in range(3):
            out_desc(jb1, 1, d).start()
        return _

    lax.fori_loop(0, (NJB - 1) // 2, pair_step, None)

    in_desc(NJB - 1, 0).wait()
    for d in range(3):
        out_desc(NJB - 3, 0, d).wait()
    compute(0)
    for d in range(3):
        out_desc(NJB - 1, 0, d).start()
    for d in range(3):
        out_desc(NJB - 2, 1, d).wait()
        out_desc(NJB - 1, 0, d).wait()


_mesh = plsc.VectorSubcoreMesh(core_axis_name="c", subcore_axis_name="s")

_sc_call = functools.partial(
    pl.kernel,
    mesh=_mesh,
    out_type=jax.ShapeDtypeStruct((2 * TOTAL_WORDS // 128, 128), jnp.float16),
    scratch_types=[
        pltpu.VMEM((2 * IBW * 1024,), jnp.int32),
        pltpu.VMEM((2 * 3 * IBW * 4, 128), jnp.int32),
        pltpu.VMEM((112,), jnp.int32),
        pltpu.VMEM((112,), jnp.int32),
        pltpu.VMEM((112,), jnp.int32),
        pltpu.VMEM((32,), jnp.int32),
        pltpu.SemaphoreType.DMA((2,)),
        pltpu.SemaphoreType.DMA((2,)),
    ],
    compiler_params=pltpu.CompilerParams(needs_layout_passes=False),
)(_body)


@jax.jit
def kernel(input_tensor, weight):
    idx_flat = (
        input_tensor.reshape(128, 128, 25, 8)
        .transpose(2, 0, 3, 1)
        .reshape(-1)
    )
    w16 = jax.lax.bitcast_convert_type(weight, jnp.uint16).reshape(-1)
    wi = jnp.zeros((32,), jnp.int32).at[:30].set(w16.astype(jnp.int32))
    out_halves = _sc_call(idx_flat, wi)
    out = (
        out_halves.reshape(3, 25, 128, 4, 2, 128)
        .transpose(2, 5, 1, 3, 4, 0)
        .reshape(B0, B1, DIM)
    )
    return out
